# Initial kernel scaffold; baseline (speedup 1.0000x reference)
import jax
import jax.numpy as jnp
from jax.experimental import pallas as pl


def kernel(output, target):
    raise NotImplementedError("write your pallas kernel here")


# speedup vs baseline: 1.5473x; 1.5473x over previous
import functools

import jax
import jax.numpy as jnp
from jax import lax
from jax.experimental import pallas as pl
from jax.experimental.pallas import tpu as pltpu
from jax.experimental.pallas import tpu_sc as plsc

N_CLASSES = 19
N_BINS = 10
H = 512
W = 512
BATCH = 8
ROWS = 128

TOTAL = BATCH * H * W
NUM_TILES = 32
PER_TILE = TOTAL // NUM_TILES
CHUNK = 4096
LANES = 16


def _dense_body(bounds_ref, x_ref, t_ref, ps_ref, key_ref):
    x0 = x_ref[0, 0]
    m = x0
    a = jnp.zeros(x0.shape, jnp.int32)
    for c in range(1, N_CLASSES):
        xc = x_ref[0, c]
        gt = xc > m
        m = jnp.where(gt, xc, m)
        a = jnp.where(gt, c, a)
    s = jnp.exp(x0 - m)
    for c in range(1, N_CLASSES):
        s = s + jnp.exp(x_ref[0, c] - m)
    ps = 1.0 / s
    correct = (a == t_ref[0]).astype(jnp.int32)
    b = jnp.zeros(ps.shape, jnp.int32)
    for i in range(1, N_BINS):
        b = b + (ps > bounds_ref[i]).astype(jnp.int32)
    ps_ref[0] = ps
    key_ref[0] = b * 2 + correct


def _dense_call(bounds, output, target):
    grid = (BATCH, H // ROWS)
    return pl.pallas_call(
        _dense_body,
        grid=grid,
        in_specs=[
            pl.BlockSpec(memory_space=pltpu.SMEM),
            pl.BlockSpec((1, N_CLASSES, ROWS, W), lambda b, r: (b, 0, r, 0)),
            pl.BlockSpec((1, ROWS, W), lambda b, r: (b, r, 0)),
        ],
        out_specs=[
            pl.BlockSpec((1, ROWS, W), lambda b, r: (b, r, 0)),
            pl.BlockSpec((1, ROWS, W), lambda b, r: (b, r, 0)),
        ],
        out_shape=[
            jax.ShapeDtypeStruct((BATCH, H, W), jnp.float32),
            jax.ShapeDtypeStruct((BATCH, H, W), jnp.int32),
        ],
        compiler_params=pltpu.CompilerParams(
            dimension_semantics=("parallel", "parallel")),
    )(bounds, output, target)


def _hist_body(ps_hbm, key_hbm, conf_out, cnt_out, ps_v, key_v, conf_acc,
               cnt_acc):
    nc = 2
    wid = lax.axis_index("s") * nc + lax.axis_index("c")
    base = wid * PER_TILE
    lanes = lax.iota(jnp.int32, LANES)
    ones = jnp.full((LANES,), 1.0, jnp.float32)
    zero16 = jnp.zeros((LANES,), jnp.float32)
    for r in range(N_BINS):
        conf_acc[pl.ds(r * LANES, LANES)] = zero16
    for r in range(2 * N_BINS):
        cnt_acc[pl.ds(r * LANES, LANES)] = zero16

    def chunk_body(i, carry):
        off = base + i * CHUNK
        pltpu.sync_copy(ps_hbm.at[pl.ds(off, CHUNK)], ps_v)
        pltpu.sync_copy(key_hbm.at[pl.ds(off, CHUNK)], key_v)

        def vec_body(j, c2):
            ps = ps_v[pl.ds(j * LANES, LANES)]
            key = key_v[pl.ds(j * LANES, LANES)]
            b = lax.shift_right_logical(key, 1)
            conf_idx = lax.shift_left(b, 4) + lanes
            cnt_idx = lax.shift_left(key, 4) + lanes
            plsc.addupdate_scatter(conf_acc, [conf_idx], ps)
            plsc.addupdate_scatter(cnt_acc, [cnt_idx], ones)
            return c2

        return lax.fori_loop(0, CHUNK // LANES, vec_body, carry)

    lax.fori_loop(0, PER_TILE // CHUNK, chunk_body, 0)
    pltpu.sync_copy(conf_acc, conf_out.at[wid])
    pltpu.sync_copy(cnt_acc, cnt_out.at[wid])


def _hist_call(ps_flat, key_flat):
    mesh = plsc.VectorSubcoreMesh(core_axis_name="c", subcore_axis_name="s",
                                  num_cores=2, num_subcores=16)
    f = pl.kernel(
        _hist_body,
        out_type=[
            jax.ShapeDtypeStruct((NUM_TILES, N_BINS * LANES), jnp.float32),
            jax.ShapeDtypeStruct((NUM_TILES, 2 * N_BINS * LANES), jnp.float32),
        ],
        mesh=mesh,
        scratch_types=[
            pltpu.VMEM((CHUNK,), jnp.float32),
            pltpu.VMEM((CHUNK,), jnp.int32),
            pltpu.VMEM((N_BINS * LANES,), jnp.float32),
            pltpu.VMEM((2 * N_BINS * LANES,), jnp.float32),
        ],
        compiler_params=pltpu.CompilerParams(needs_layout_passes=False),
    )
    return f(ps_flat, key_flat)


def _final_body(conf_ref, cnt_ref, out_ref):
    conf_t = jnp.sum(conf_ref[...], axis=0)
    cnt_t = jnp.sum(cnt_ref[...], axis=0)
    ns = []
    accs = []
    confs = []
    for b in range(N_BINS):
        n0 = jnp.sum(cnt_t[2 * b])
        n1 = jnp.sum(cnt_t[2 * b + 1])
        ns.append(n0 + n1)
        accs.append(n1)
        confs.append(jnp.sum(conf_t[b]))
    total = ns[0]
    for b in range(1, N_BINS):
        total = total + ns[b]
    ece = jnp.float32(0.0)
    for b in range(N_BINS):
        denom = ns[b] + 1e-13
        avg_acc = accs[b] / denom
        avg_conf = confs[b] / denom
        diff = jnp.abs(avg_acc - avg_conf)
        ece = ece + diff * diff * (ns[b] / total)
    out_ref[0, 0] = ece


def _final_call(conf, cnt):
    return pl.pallas_call(
        _final_body,
        out_specs=pl.BlockSpec(memory_space=pltpu.SMEM),
        out_shape=jax.ShapeDtypeStruct((1, 1), jnp.float32),
    )(conf, cnt)


def kernel(output, target):
    target = target.astype(jnp.int32)
    bounds = jnp.linspace(0.0, 1.0, N_BINS + 1).astype(jnp.float32)
    ps, key = _dense_call(bounds, output, target)
    conf, cnt = _hist_call(ps.reshape(-1), key.reshape(-1))
    ece = _final_call(conf.reshape(NUM_TILES, N_BINS, LANES),
                      cnt.reshape(NUM_TILES, 2 * N_BINS, LANES))
    return ece[0, 0]


# speedup vs baseline: 2.8309x; 1.8296x over previous
import jax
import jax.numpy as jnp
from jax import lax
from jax.experimental import pallas as pl
from jax.experimental.pallas import tpu as pltpu
from jax.experimental.pallas import tpu_sc as plsc

N_CLASSES = 19
N_BINS = 10
H = 512
W = 512
BATCH = 8
ROWS = 128

FLAT_ROWS = BATCH * H
NUM_TILES = 32
TILE_ROWS = FLAT_ROWS // NUM_TILES
CHUNK_ROWS = 32
LANES = 16


def _dense_body(x_ref, t_ref, out_ref):
    t = t_ref[0]
    x0 = x_ref[0, 0]
    m = x0
    tv = x0
    s = jnp.exp(x0)
    for c in range(1, N_CLASSES):
        xc = x_ref[0, c]
        m = jnp.maximum(m, xc)
        tv = jnp.where(t == c, xc, tv)
        s = s + jnp.exp(xc)
    ps = jnp.exp(m) / s
    correct = (tv == m).astype(jnp.int32)
    b = jnp.minimum(lax.convert_element_type(ps * 10.0, jnp.int32), 9)
    key = b * 2 + correct
    packed = lax.bitcast_convert_type(
        (lax.bitcast_convert_type(ps, jnp.int32) & -32) | key, jnp.float32)
    out_ref[...] = packed


def _dense_call(output, target):
    grid = (BATCH, H // ROWS)
    rsteps = H // ROWS
    return pl.pallas_call(
        _dense_body,
        grid=grid,
        in_specs=[
            pl.BlockSpec((1, N_CLASSES, ROWS, W), lambda b, r: (b, 0, r, 0)),
            pl.BlockSpec((1, ROWS, W), lambda b, r: (b, r, 0)),
        ],
        out_specs=pl.BlockSpec((ROWS, W), lambda b, r: (b * rsteps + r, 0)),
        out_shape=jax.ShapeDtypeStruct((FLAT_ROWS, W), jnp.float32),
        compiler_params=pltpu.CompilerParams(
            dimension_semantics=("parallel", "parallel")),
    )(output, target)


def _hist_body(packed_hbm, conf_out, cnt_out, buf, conf_acc, cnt_acc):
    nc = 2
    wid = lax.axis_index("s") * nc + lax.axis_index("c")
    row0 = wid * TILE_ROWS
    lanes = lax.iota(jnp.int32, LANES)
    ones = jnp.full((LANES,), 1.0, jnp.float32)
    zero16 = jnp.zeros((LANES,), jnp.float32)
    for r in range(N_BINS):
        conf_acc[pl.ds(r * LANES, LANES)] = zero16
    for r in range(2 * N_BINS):
        cnt_acc[pl.ds(r * LANES, LANES)] = zero16

    vregs_per_chunk = CHUNK_ROWS * W // LANES

    def chunk_body(i, carry):
        pltpu.sync_copy(packed_hbm.at[pl.ds(row0 + i * CHUNK_ROWS,
                                            CHUNK_ROWS)], buf)

        @plsc.parallel_loop(0, vregs_per_chunk, unroll=8)
        def vec_body(j):
            row = lax.shift_right_logical(j, 5)
            col = lax.shift_left(lax.bitwise_and(j, 31), 4)
            v = buf[row, pl.ds(col, LANES)]
            vi = plsc.bitcast(v, jnp.int32)
            key = lax.bitwise_and(vi, 31)
            b = lax.shift_right_logical(key, 1)
            conf_idx = lax.shift_left(b, 4) + lanes
            cnt_idx = lax.shift_left(key, 4) + lanes
            plsc.addupdate_scatter(conf_acc, [conf_idx], v)
            plsc.addupdate_scatter(cnt_acc, [cnt_idx], ones)

        return carry

    lax.fori_loop(0, TILE_ROWS // CHUNK_ROWS, chunk_body, 0)
    pltpu.sync_copy(conf_acc, conf_out.at[wid])
    pltpu.sync_copy(cnt_acc, cnt_out.at[wid])


def _hist_call(packed):
    mesh = plsc.VectorSubcoreMesh(core_axis_name="c", subcore_axis_name="s",
                                  num_cores=2, num_subcores=16)
    f = pl.kernel(
        _hist_body,
        out_type=[
            jax.ShapeDtypeStruct((NUM_TILES, N_BINS * LANES), jnp.float32),
            jax.ShapeDtypeStruct((NUM_TILES, 2 * N_BINS * LANES), jnp.float32),
        ],
        mesh=mesh,
        scratch_types=[
            pltpu.VMEM((CHUNK_ROWS, W), jnp.float32),
            pltpu.VMEM((N_BINS * LANES,), jnp.float32),
            pltpu.VMEM((2 * N_BINS * LANES,), jnp.float32),
        ],
        compiler_params=pltpu.CompilerParams(needs_layout_passes=False),
    )
    return f(packed)


def _final_body(conf_ref, cnt_ref, out_ref):
    conf_t = jnp.sum(conf_ref[...], axis=0)
    cnt_t = jnp.sum(cnt_ref[...], axis=0)
    ns = []
    accs = []
    confs = []
    for b in range(N_BINS):
        n0 = jnp.sum(cnt_t[2 * b * LANES:(2 * b + 1) * LANES])
        n1 = jnp.sum(cnt_t[(2 * b + 1) * LANES:(2 * b + 2) * LANES])
        ns.append(n0 + n1)
        accs.append(n1)
        confs.append(jnp.sum(conf_t[b * LANES:(b + 1) * LANES]))
    total = ns[0]
    for b in range(1, N_BINS):
        total = total + ns[b]
    ece = jnp.float32(0.0)
    for b in range(N_BINS):
        denom = ns[b] + 1e-13
        avg_acc = accs[b] / denom
        avg_conf = confs[b] / denom
        diff = jnp.abs(avg_acc - avg_conf)
        ece = ece + diff * diff * (ns[b] / total)
    out_ref[0, 0] = ece


def _final_call(conf, cnt):
    return pl.pallas_call(
        _final_body,
        out_specs=pl.BlockSpec(memory_space=pltpu.SMEM),
        out_shape=jax.ShapeDtypeStruct((1, 1), jnp.float32),
    )(conf, cnt)


def kernel(output, target):
    target = target.astype(jnp.int32)
    packed = _dense_call(output, target)
    conf, cnt = _hist_call(packed)
    ece = _final_call(conf, cnt)
    return ece[0, 0]


# speedup vs baseline: 2.8503x; 1.0069x over previous
import jax
import jax.numpy as jnp
from jax import lax
from jax.experimental import pallas as pl
from jax.experimental.pallas import tpu as pltpu
from jax.experimental.pallas import tpu_sc as plsc

N_CLASSES = 19
N_BINS = 10
H = 512
W = 512
BATCH = 8
ROWS = 128
SLICES = 4
NB = BATCH // SLICES

SLICE_ROWS = NB * H
NUM_TILES = 32
TILE_ROWS = SLICE_ROWS // NUM_TILES
LANES = 16


def _dense_body(x_ref, t_ref, out_ref):
    t = t_ref[0]
    x0 = x_ref[0, 0]
    m = x0
    tv = x0
    s = jnp.exp(x0)
    for c in range(1, N_CLASSES):
        xc = x_ref[0, c]
        m = jnp.maximum(m, xc)
        tv = jnp.where(t == c, xc, tv)
        s = s + jnp.exp(xc)
    ps = jnp.exp(m) / s
    correct = (tv == m).astype(jnp.int32)
    b = jnp.minimum(lax.convert_element_type(ps * 10.0, jnp.int32), 9)
    key = b * 2 + correct
    packed = lax.bitcast_convert_type(
        (lax.bitcast_convert_type(ps, jnp.int32) & -32) | key, jnp.float32)
    out_ref[...] = packed


def _dense_call(output, target, k):
    rsteps = H // ROWS
    return pl.pallas_call(
        _dense_body,
        grid=(NB, rsteps),
        in_specs=[
            pl.BlockSpec((1, N_CLASSES, ROWS, W),
                         lambda b, r, k=k: (k * NB + b, 0, r, 0)),
            pl.BlockSpec((1, ROWS, W), lambda b, r, k=k: (k * NB + b, r, 0)),
        ],
        out_specs=pl.BlockSpec((ROWS, W), lambda b, r: (b * rsteps + r, 0)),
        out_shape=jax.ShapeDtypeStruct((SLICE_ROWS, W), jnp.float32),
        compiler_params=pltpu.CompilerParams(
            dimension_semantics=("parallel", "parallel")),
    )(output, target)


def _hist_body(packed_hbm, conf_out, cnt_out, buf, conf_acc, cnt_acc):
    nc = 2
    wid = lax.axis_index("s") * nc + lax.axis_index("c")
    lanes = lax.iota(jnp.int32, LANES)
    ones = jnp.full((LANES,), 1.0, jnp.float32)
    zero16 = jnp.zeros((LANES,), jnp.float32)
    for r in range(N_BINS):
        conf_acc[pl.ds(r * LANES, LANES)] = zero16
    for r in range(2 * N_BINS):
        cnt_acc[pl.ds(r * LANES, LANES)] = zero16

    pltpu.sync_copy(packed_hbm.at[pl.ds(wid * TILE_ROWS, TILE_ROWS)], buf)

    @plsc.parallel_loop(0, TILE_ROWS * W // LANES, unroll=8)
    def vec_body(j):
        row = lax.shift_right_logical(j, 5)
        col = lax.shift_left(lax.bitwise_and(j, 31), 4)
        v = buf[row, pl.ds(col, LANES)]
        vi = plsc.bitcast(v, jnp.int32)
        key = lax.bitwise_and(vi, 31)
        b = lax.shift_right_logical(key, 1)
        conf_idx = lax.shift_left(b, 4) + lanes
        cnt_idx = lax.shift_left(key, 4) + lanes
        plsc.addupdate_scatter(conf_acc, [conf_idx], v)
        plsc.addupdate_scatter(cnt_acc, [cnt_idx], ones)

    pltpu.sync_copy(conf_acc, conf_out.at[wid])
    pltpu.sync_copy(cnt_acc, cnt_out.at[wid])


def _hist_call(packed):
    mesh = plsc.VectorSubcoreMesh(core_axis_name="c", subcore_axis_name="s",
                                  num_cores=2, num_subcores=16)
    f = pl.kernel(
        _hist_body,
        out_type=[
            jax.ShapeDtypeStruct((NUM_TILES, N_BINS * LANES), jnp.float32),
            jax.ShapeDtypeStruct((NUM_TILES, 2 * N_BINS * LANES), jnp.float32),
        ],
        mesh=mesh,
        scratch_types=[
            pltpu.VMEM((TILE_ROWS, W), jnp.float32),
            pltpu.VMEM((N_BINS * LANES,), jnp.float32),
            pltpu.VMEM((2 * N_BINS * LANES,), jnp.float32),
        ],
        compiler_params=pltpu.CompilerParams(needs_layout_passes=False),
    )
    return f(packed)


def _final_body(*refs):
    conf_refs = refs[:SLICES]
    cnt_refs = refs[SLICES:2 * SLICES]
    out_ref = refs[2 * SLICES]
    conf_t = jnp.sum(conf_refs[0][...], axis=0)
    cnt_t = jnp.sum(cnt_refs[0][...], axis=0)
    for k in range(1, SLICES):
        conf_t = conf_t + jnp.sum(conf_refs[k][...], axis=0)
        cnt_t = cnt_t + jnp.sum(cnt_refs[k][...], axis=0)
    ns = []
    accs = []
    confs = []
    for b in range(N_BINS):
        n0 = jnp.sum(cnt_t[2 * b * LANES:(2 * b + 1) * LANES])
        n1 = jnp.sum(cnt_t[(2 * b + 1) * LANES:(2 * b + 2) * LANES])
        ns.append(n0 + n1)
        accs.append(n1)
        confs.append(jnp.sum(conf_t[b * LANES:(b + 1) * LANES]))
    total = ns[0]
    for b in range(1, N_BINS):
        total = total + ns[b]
    ece = jnp.float32(0.0)
    for b in range(N_BINS):
        denom = ns[b] + 1e-13
        avg_acc = accs[b] / denom
        avg_conf = confs[b] / denom
        diff = jnp.abs(avg_acc - avg_conf)
        ece = ece + diff * diff * (ns[b] / total)
    out_ref[0, 0] = ece


def _final_call(confs, cnts):
    return pl.pallas_call(
        _final_body,
        out_specs=pl.BlockSpec(memory_space=pltpu.SMEM),
        out_shape=jax.ShapeDtypeStruct((1, 1), jnp.float32),
    )(*confs, *cnts)


def kernel(output, target):
    target = target.astype(jnp.int32)
    confs = []
    cnts = []
    for k in range(SLICES):
        packed = _dense_call(output, target, k)
        conf, cnt = _hist_call(packed)
        confs.append(conf)
        cnts.append(cnt)
    ece = _final_call(confs, cnts)
    return ece[0, 0]
